# baseline (speedup 1.0000x reference)
import math

import jax
import jax.numpy as jnp
from jax.experimental import pallas as pl
from jax.experimental.pallas import tpu as pltpu

_K = 5
_N = 4096
_D_IN = 512
_D_HID = 256
_D_OUT = 64
_BP = 256
_RP = _N // _BP
_BR = 1024
_RB = _N // _BR
_HK = _N // 4


def _split128(u):
    uh = u.astype(jnp.bfloat16)
    ul = (u - uh.astype(jnp.float32)).astype(jnp.bfloat16)
    return jnp.concatenate([uh, ul], axis=1)


def _prep_body(fp_ref, comb_ref, x_ref, w1_ref, b1_ref, w2_ref, b2_ref,
               p_ref, p16_ref, vb_ref, tb_ref):
    p16_ref[...] = p_ref[...].astype(jnp.bfloat16)
    h1 = jnp.dot(x_ref[...], w1_ref[...], preferred_element_type=jnp.float32)
    h1 = jnp.maximum(h1 + b1_ref[...], 0.0)
    h = jnp.dot(h1, w2_ref[...], preferred_element_type=jnp.float32) + b2_ref[...]
    cK = jnp.maximum(fp_ref[_K, 0], 0.0) * comb_ref[_K, 0]
    vb_ref[...] = _split128(h)
    tb_ref[...] = _split128(cK * h)


def _main_body(fp_ref, comb_ref, a_ref, p16_ref, vb0_ref, tb0_ref, out_ref,
               a_scr, va, ta, vb, tb, acc_v, acc_t):
    s = pl.program_id(0)
    r = pl.program_id(1)
    c = pl.program_id(2)
    row = r * _BR
    col = c * _HK

    @pl.when(s == 0)
    def _park():
        a_scr[pl.ds(row, _BR), pl.ds(col, _HK)] = a_ref[...].astype(jnp.bfloat16)

    def _step(svb, stb, dvb, dtb):
        pv = jnp.dot(a_scr[pl.ds(row, _BR), pl.ds(col, _HK)],
                     svb[pl.ds(col, _HK), :], preferred_element_type=jnp.float32)
        pt = jnp.dot(p16_ref[...], stb[pl.ds(col, _HK), :],
                     preferred_element_type=jnp.float32)

        @pl.when(c == 0)
        def _acc0():
            acc_v[...] = pv
            acc_t[...] = pt

        @pl.when((c > 0) & (c < 3))
        def _accm():
            acc_v[...] = acc_v[...] + pv
            acc_t[...] = acc_t[...] + pt

        @pl.when(c == 3)
        def _fin():
            i = _K - s
            coef = jnp.maximum(fp_ref[i, 0], 0.0) * comb_ref[i, 0]
            v_pair = acc_v[...] + pv
            v_new = v_pair[:, :_D_OUT] + v_pair[:, _D_OUT:]
            t_pair = acc_t[...] + pt
            t_new = t_pair[:, :_D_OUT] + t_pair[:, _D_OUT:] + coef * v_new
            dvb[pl.ds(row, _BR), :] = _split128(v_new)
            dtb[pl.ds(row, _BR), :] = _split128(t_new)

            @pl.when(s == _K)
            def _out():
                m = jnp.max(t_new, axis=1, keepdims=True)
                lse = jnp.log(jnp.sum(jnp.exp(t_new - m), axis=1, keepdims=True)) + m
                out_ref[...] = t_new - lse

    @pl.when(s == 1)
    def _first():
        _step(vb0_ref, tb0_ref, va, ta)

    @pl.when((s > 1) & (s % 2 == 0))
    def _even():
        _step(va, ta, vb, tb)

    @pl.when((s > 1) & (s % 2 == 1))
    def _odd():
        _step(vb, tb, va, ta)


def kernel(x, adj, poly_item, W1, b1, W2, b2, filter_param):
    comb = jnp.asarray(
        [[math.comb(_K, i) / (2.0 ** _K)] for i in range(_K + 1)], dtype=jnp.float32)
    b1r = b1.reshape(1, _D_HID)
    b2r = b2.reshape(1, _D_OUT)

    p16, vb0, tb0 = pl.pallas_call(
        _prep_body,
        grid=(_RP,),
        in_specs=[
            pl.BlockSpec(memory_space=pltpu.SMEM),
            pl.BlockSpec(memory_space=pltpu.SMEM),
            pl.BlockSpec((_BP, _D_IN), lambda r: (r, 0)),
            pl.BlockSpec((_D_IN, _D_HID), lambda r: (0, 0)),
            pl.BlockSpec((1, _D_HID), lambda r: (0, 0)),
            pl.BlockSpec((_D_HID, _D_OUT), lambda r: (0, 0)),
            pl.BlockSpec((1, _D_OUT), lambda r: (0, 0)),
            pl.BlockSpec((_BP, _N), lambda r: (r, 0)),
        ],
        out_specs=[
            pl.BlockSpec((_BP, _N), lambda r: (r, 0)),
            pl.BlockSpec((_BP, 2 * _D_OUT), lambda r: (r, 0)),
            pl.BlockSpec((_BP, 2 * _D_OUT), lambda r: (r, 0)),
        ],
        out_shape=[
            jax.ShapeDtypeStruct((_N, _N), jnp.bfloat16),
            jax.ShapeDtypeStruct((_N, 2 * _D_OUT), jnp.bfloat16),
            jax.ShapeDtypeStruct((_N, 2 * _D_OUT), jnp.bfloat16),
        ],
        compiler_params=pltpu.CompilerParams(
            dimension_semantics=("arbitrary",),
        ),
    )(filter_param, comb, x, W1, b1r, W2, b2r, poly_item)

    out = pl.pallas_call(
        _main_body,
        grid=(_K + 1, _RB, 4),
        in_specs=[
            pl.BlockSpec(memory_space=pltpu.SMEM),
            pl.BlockSpec(memory_space=pltpu.SMEM),
            pl.BlockSpec((_BR, _HK),
                         lambda s, r, c: (jnp.where(s == 0, r, _RB - 1),
                                          jnp.where(s == 0, c, 3))),
            pl.BlockSpec((_BR, _HK),
                         lambda s, r, c: (jnp.where(s == 0, 0, r),
                                          jnp.where(s == 0, 0, c))),
            pl.BlockSpec((_N, 2 * _D_OUT), lambda s, r, c: (0, 0)),
            pl.BlockSpec((_N, 2 * _D_OUT), lambda s, r, c: (0, 0)),
        ],
        out_specs=pl.BlockSpec(
            (_BR, _D_OUT), lambda s, r, c: (jnp.where(s == _K, r, 0), 0)),
        out_shape=jax.ShapeDtypeStruct((_N, _D_OUT), jnp.float32),
        scratch_shapes=[
            pltpu.VMEM((_N, _N), jnp.bfloat16),
            pltpu.VMEM((_N, 2 * _D_OUT), jnp.bfloat16),
            pltpu.VMEM((_N, 2 * _D_OUT), jnp.bfloat16),
            pltpu.VMEM((_N, 2 * _D_OUT), jnp.bfloat16),
            pltpu.VMEM((_N, 2 * _D_OUT), jnp.bfloat16),
            pltpu.VMEM((_BR, 2 * _D_OUT), jnp.float32),
            pltpu.VMEM((_BR, 2 * _D_OUT), jnp.float32),
        ],
        compiler_params=pltpu.CompilerParams(
            dimension_semantics=("arbitrary", "arbitrary", "arbitrary"),
        ),
    )(filter_param, comb, adj, p16, vb0, tb0)
    return out


# speedup vs baseline: 1.1029x; 1.1029x over previous
import math

import jax
import jax.numpy as jnp
from jax.experimental import pallas as pl
from jax.experimental.pallas import tpu as pltpu

_K = 5
_N = 4096
_D_IN = 512
_D_HID = 256
_D_OUT = 64
_BP = 256
_RP = _N // _BP
_BR = 1024
_RB = _N // _BR


def _split128(u):
    uh = u.astype(jnp.bfloat16)
    ul = (u - uh.astype(jnp.float32)).astype(jnp.bfloat16)
    return jnp.concatenate([uh, ul], axis=1)


def _prep_body(fp_ref, comb_ref, x_ref, w1_ref, b1_ref, w2_ref, b2_ref,
               a_ref, p_ref, a16_ref, p16_ref, vb_ref, tb_ref):
    a16_ref[...] = a_ref[...].astype(jnp.bfloat16)
    p16_ref[...] = p_ref[...].astype(jnp.bfloat16)
    h1 = jnp.dot(x_ref[...], w1_ref[...], preferred_element_type=jnp.float32)
    h1 = jnp.maximum(h1 + b1_ref[...], 0.0)
    h = jnp.dot(h1, w2_ref[...], preferred_element_type=jnp.float32) + b2_ref[...]
    cK = jnp.maximum(fp_ref[_K, 0], 0.0) * comb_ref[_K, 0]
    vb_ref[...] = _split128(h)
    tb_ref[...] = _split128(cK * h)


def _main_body(fp_ref, comb_ref, a16_ref, p16_ref, vb0_ref, tb0_ref, out_ref,
               va, ta, vb, tb):
    s = pl.program_id(0)
    r = pl.program_id(1)
    row = r * _BR

    def _step(svb, stb, dvb, dtb):
        i = _K - 1 - s
        coef = jnp.maximum(fp_ref[i, 0], 0.0) * comb_ref[i, 0]
        v_pair = jnp.dot(a16_ref[pl.ds(row, _BR), :], svb[...],
                         preferred_element_type=jnp.float32)
        v_new = v_pair[:, :_D_OUT] + v_pair[:, _D_OUT:]
        t_pair = jnp.dot(p16_ref[...], stb[...],
                         preferred_element_type=jnp.float32)
        t_new = t_pair[:, :_D_OUT] + t_pair[:, _D_OUT:] + coef * v_new
        dvb[pl.ds(row, _BR), :] = _split128(v_new)
        dtb[pl.ds(row, _BR), :] = _split128(t_new)

        @pl.when(s == _K - 1)
        def _out():
            m = jnp.max(t_new, axis=1, keepdims=True)
            lse = jnp.log(jnp.sum(jnp.exp(t_new - m), axis=1, keepdims=True)) + m
            out_ref[...] = t_new - lse

    @pl.when(s == 0)
    def _first():
        _step(vb0_ref, tb0_ref, va, ta)

    @pl.when((s > 0) & (s % 2 == 1))
    def _odd():
        _step(va, ta, vb, tb)

    @pl.when((s > 0) & (s % 2 == 0))
    def _even():
        _step(vb, tb, va, ta)


def kernel(x, adj, poly_item, W1, b1, W2, b2, filter_param):
    comb = jnp.asarray(
        [[math.comb(_K, i) / (2.0 ** _K)] for i in range(_K + 1)], dtype=jnp.float32)
    b1r = b1.reshape(1, _D_HID)
    b2r = b2.reshape(1, _D_OUT)

    a16, p16, vb0, tb0 = pl.pallas_call(
        _prep_body,
        grid=(_RP,),
        in_specs=[
            pl.BlockSpec(memory_space=pltpu.SMEM),
            pl.BlockSpec(memory_space=pltpu.SMEM),
            pl.BlockSpec((_BP, _D_IN), lambda r: (r, 0)),
            pl.BlockSpec((_D_IN, _D_HID), lambda r: (0, 0)),
            pl.BlockSpec((1, _D_HID), lambda r: (0, 0)),
            pl.BlockSpec((_D_HID, _D_OUT), lambda r: (0, 0)),
            pl.BlockSpec((1, _D_OUT), lambda r: (0, 0)),
            pl.BlockSpec((_BP, _N), lambda r: (r, 0)),
            pl.BlockSpec((_BP, _N), lambda r: (r, 0)),
        ],
        out_specs=[
            pl.BlockSpec((_BP, _N), lambda r: (r, 0)),
            pl.BlockSpec((_BP, _N), lambda r: (r, 0)),
            pl.BlockSpec((_BP, 2 * _D_OUT), lambda r: (r, 0)),
            pl.BlockSpec((_BP, 2 * _D_OUT), lambda r: (r, 0)),
        ],
        out_shape=[
            jax.ShapeDtypeStruct((_N, _N), jnp.bfloat16),
            jax.ShapeDtypeStruct((_N, _N), jnp.bfloat16),
            jax.ShapeDtypeStruct((_N, 2 * _D_OUT), jnp.bfloat16),
            jax.ShapeDtypeStruct((_N, 2 * _D_OUT), jnp.bfloat16),
        ],
        compiler_params=pltpu.CompilerParams(
            dimension_semantics=("arbitrary",),
        ),
    )(filter_param, comb, x, W1, b1r, W2, b2r, adj, poly_item)

    out = pl.pallas_call(
        _main_body,
        grid=(_K, _RB),
        in_specs=[
            pl.BlockSpec(memory_space=pltpu.SMEM),
            pl.BlockSpec(memory_space=pltpu.SMEM),
            pl.BlockSpec((_N, _N), lambda s, r: (0, 0)),
            pl.BlockSpec((_BR, _N), lambda s, r: (r, 0)),
            pl.BlockSpec((_N, 2 * _D_OUT), lambda s, r: (0, 0)),
            pl.BlockSpec((_N, 2 * _D_OUT), lambda s, r: (0, 0)),
        ],
        out_specs=pl.BlockSpec((_BR, _D_OUT),
                               lambda s, r: (jnp.where(s == _K - 1, r, 0), 0)),
        out_shape=jax.ShapeDtypeStruct((_N, _D_OUT), jnp.float32),
        scratch_shapes=[
            pltpu.VMEM((_N, 2 * _D_OUT), jnp.bfloat16),
            pltpu.VMEM((_N, 2 * _D_OUT), jnp.bfloat16),
            pltpu.VMEM((_N, 2 * _D_OUT), jnp.bfloat16),
            pltpu.VMEM((_N, 2 * _D_OUT), jnp.bfloat16),
        ],
        compiler_params=pltpu.CompilerParams(
            dimension_semantics=("arbitrary", "arbitrary"),
        ),
    )(filter_param, comb, a16, p16, vb0, tb0)
    return out
